# baseline (speedup 1.0000x reference)
import functools
import jax
import jax.numpy as jnp
from jax import lax
from jax.experimental import pallas as pl
from jax.experimental.pallas import tpu as pltpu, tpu_sc as plsc

C = 256
H = 100
W = 100
Z = 8
NW = 32
PJ = 10
NCH = W // PJ
NR = 4

_mesh = plsc.VectorSubcoreMesh(core_axis_name="c", subcore_axis_name="s")


@functools.partial(
    pl.kernel,
    mesh=_mesh,
    compiler_params=pltpu.CompilerParams(needs_layout_passes=False),
    out_type=jax.ShapeDtypeStruct((H, W, Z, C), jnp.float32),
    scratch_types=[
        pltpu.VMEM((Z * C,), jnp.float32),
        pltpu.VMEM((C,), jnp.float32),
        pltpu.VMEM((PJ * C,), jnp.float32),
        pltpu.VMEM((4, PJ, Z, C), jnp.float32),
        pltpu.SemaphoreType.DMA,
    ],
)
def _sc_kernel(col_hbm, row_hbm, z_hbm, out_hbm, zvm, colv, rwv, buf, sem):
    wid = lax.axis_index("s") * 2 + lax.axis_index("c")
    pltpu.sync_copy(z_hbm, zvm)

    for r in range(NR):
        i = wid + NW * r

        @pl.when(i < H)
        def _row():
            pltpu.sync_copy(col_hbm.at[pl.ds(i * C, C)], colv)

            def chunk(u, carry):
                b = lax.rem(u, 4)

                @pl.when(u >= 4)
                def _drain():
                    pltpu.make_async_copy(
                        buf.at[b], out_hbm.at[i, pl.ds(u * PJ, PJ)], sem
                    ).wait()

                pltpu.sync_copy(row_hbm.at[pl.ds(u * (PJ * C), PJ * C)], rwv)

                for t in range(C // 16):
                    colv_t = colv[pl.ds(16 * t, 16)]
                    zk = [zvm[pl.ds(k * C + 16 * t, 16)] + colv_t
                          for k in range(Z)]

                    def fill(p, carry2):
                        rw = rwv[pl.ds(p * C + 16 * t, 16)]
                        for k in range(Z):
                            buf[b, p, k, pl.ds(16 * t, 16)] = zk[k] + rw
                        return carry2

                    lax.fori_loop(0, PJ, fill, 0)
                pltpu.async_copy(
                    buf.at[b], out_hbm.at[i, pl.ds(u * PJ, PJ)], sem
                )
                return carry

            lax.fori_loop(0, NCH, chunk, 0)
            for d in range(4):
                pltpu.make_async_copy(
                    buf.at[d], out_hbm.at[i, pl.ds(d * PJ, PJ)], sem
                ).wait()


def kernel(row_weight, col_weight, z_weight, bs, h, w, z):
    pos4 = _sc_kernel(
        col_weight.reshape(-1), row_weight.reshape(-1), z_weight.reshape(-1)
    )
    return jnp.transpose(pos4, (3, 0, 1, 2))[None]


# speedup vs baseline: 1.7000x; 1.7000x over previous
import functools
import jax
import jax.numpy as jnp
from jax import lax
from jax.experimental import pallas as pl
from jax.experimental.pallas import tpu as pltpu, tpu_sc as plsc

C = 256
H = 100
W = 100
Z = 8
NW = 32
PJ = 20
NCH = W // PJ
NR = 4

_mesh = plsc.VectorSubcoreMesh(core_axis_name="c", subcore_axis_name="s")


@functools.partial(
    pl.kernel,
    mesh=_mesh,
    compiler_params=pltpu.CompilerParams(needs_layout_passes=False),
    out_type=jax.ShapeDtypeStruct((H, W, Z, C), jnp.float32),
    scratch_types=[
        pltpu.VMEM((Z * C,), jnp.float32),
        pltpu.VMEM((C,), jnp.float32),
        pltpu.VMEM((2, PJ * C), jnp.float32),
        pltpu.VMEM((2, PJ, Z, C), jnp.float32),
        pltpu.SemaphoreType.DMA,
        pltpu.SemaphoreType.DMA,
    ],
)
def _sc_kernel(col_hbm, row_hbm, z_hbm, out_hbm, zvm, colv, rwv, buf, sem, sem_rw):
    wid = lax.axis_index("s") * 2 + lax.axis_index("c")
    pltpu.sync_copy(z_hbm, zvm)

    def rw_slice(u, m):
        return (row_hbm.at[pl.ds(u * (PJ * C), PJ * C)], rwv.at[m])

    for r in range(NR):
        i = wid + NW * r

        @pl.when(i < H)
        def _row():
            pltpu.sync_copy(col_hbm.at[pl.ds(i * C, C)], colv)
            s0, d0 = rw_slice(0, 0)
            pltpu.sync_copy(s0, d0)

            def chunk(u, carry):
                b = lax.rem(u, 2)
                m = lax.rem(u, 2)

                @pl.when(u >= 2)
                def _drain():
                    pltpu.make_async_copy(
                        buf.at[b], out_hbm.at[i, pl.ds(u * PJ, PJ)], sem
                    ).wait()

                @pl.when(u + 1 < NCH)
                def _prefetch():
                    s, dref = rw_slice(u + 1, lax.rem(u + 1, 2))
                    pltpu.async_copy(s, dref, sem_rw)

                @pl.when(u >= 1)
                def _wait_rw():
                    s, dref = rw_slice(u, m)
                    pltpu.make_async_copy(s, dref, sem_rw).wait()

                for t in range(C // 16):
                    colv_t = colv[pl.ds(16 * t, 16)]
                    zk = [zvm[pl.ds(k * C + 16 * t, 16)] + colv_t
                          for k in range(Z)]

                    def fill(p, carry2):
                        rw = rwv[m, pl.ds(p * C + 16 * t, 16)]
                        for k in range(Z):
                            buf[b, p, k, pl.ds(16 * t, 16)] = zk[k] + rw
                        return carry2

                    lax.fori_loop(0, PJ, fill, 0)

                pltpu.async_copy(
                    buf.at[b], out_hbm.at[i, pl.ds(u * PJ, PJ)], sem
                )
                return carry

            lax.fori_loop(0, NCH, chunk, 0)
            pltpu.make_async_copy(buf.at[0], out_hbm.at[i, pl.ds(0, PJ)], sem).wait()
            pltpu.make_async_copy(buf.at[1], out_hbm.at[i, pl.ds(PJ, PJ)], sem).wait()


def kernel(row_weight, col_weight, z_weight, bs, h, w, z):
    pos4 = _sc_kernel(
        col_weight.reshape(-1), row_weight.reshape(-1), z_weight.reshape(-1)
    )
    return jnp.transpose(pos4, (3, 0, 1, 2))[None]


# speedup vs baseline: 6.9804x; 4.1061x over previous
import jax
import jax.numpy as jnp
from jax.experimental import pallas as pl

C = 256
H = 100
W = 100
Z = 8
H_BLK = 8


def _body(col_ref, row_ref, z_ref, out_ref):
    col_b = col_ref[...]
    row_b = row_ref[...]
    z_b = z_ref[...]
    out_ref[...] = (
        col_b[:, None, None, :] + row_b[None, :, None, :] + z_b[None, None, :, :]
    )


def kernel(row_weight, col_weight, z_weight, bs, h, w, z):
    pos4 = pl.pallas_call(
        _body,
        grid=(pl.cdiv(H, H_BLK),),
        in_specs=[
            pl.BlockSpec((H_BLK, C), lambda i: (i, 0)),
            pl.BlockSpec((W, C), lambda i: (0, 0)),
            pl.BlockSpec((Z, C), lambda i: (0, 0)),
        ],
        out_specs=pl.BlockSpec((H_BLK, W, Z, C), lambda i: (i, 0, 0, 0)),
        out_shape=jax.ShapeDtypeStruct((H, W, Z, C), jnp.float32),
    )(col_weight, row_weight, z_weight)
    return jnp.transpose(pos4, (3, 0, 1, 2))[None]
